# Initial kernel scaffold; baseline (speedup 1.0000x reference)
import jax
import jax.numpy as jnp
from jax.experimental import pallas as pl


def kernel(idx, targets, token_table, pos_table, W, b):
    raise NotImplementedError("write your pallas kernel here")


# speedup vs baseline: 1.4037x; 1.4037x over previous
import functools

import jax
import jax.numpy as jnp
from jax import lax
from jax.experimental import pallas as pl
from jax.experimental.pallas import tpu as pltpu
from jax.experimental.pallas import tpu_sc as plsc

V = 1000
NC = 2
NS = 16
NW = NC * NS
L = 16
CH = 64


def _tc_tables_body(tt_ref, w_ref, b_ref, m_ref, a_ref):
    m = jnp.dot(tt_ref[...], w_ref[...], preferred_element_type=jnp.float32)
    m = m + b_ref[...]
    m_ref[...] = m
    mx = jnp.max(m, axis=1, keepdims=True)
    s = jnp.sum(jnp.exp(m - mx), axis=1, keepdims=True)
    a_ref[...] = (mx + jnp.log(s)) - m


def _tc_tables(token_table, w, b):
    return pl.pallas_call(
        _tc_tables_body,
        out_shape=(
            jax.ShapeDtypeStruct((V, V), jnp.float32),
            jax.ShapeDtypeStruct((V, V), jnp.float32),
        ),
    )(token_table, w, b.reshape(1, V))


@functools.lru_cache(maxsize=None)
def _make_sc_gather(n_rows):
    b_per_w = n_rows // NW
    n_ch = b_per_w // CH
    mesh = plsc.VectorSubcoreMesh(core_axis_name="c", subcore_axis_name="s")

    @functools.partial(
        pl.kernel,
        out_type=(
            jax.ShapeDtypeStruct((n_rows, V), jnp.float32),
            jax.ShapeDtypeStruct((NW, L), jnp.float32),
        ),
        mesh=mesh,
        compiler_params=pltpu.CompilerParams(use_tc_tiling_on_sc=False),
        scratch_types=[
            pltpu.VMEM((b_per_w,), jnp.int32),
            pltpu.VMEM((b_per_w,), jnp.int32),
            pltpu.VMEM((CH,), jnp.int32),
            pltpu.VMEM((CH,), jnp.float32),
            pltpu.VMEM((CH, V), jnp.float32),
            pltpu.VMEM((L,), jnp.float32),
            pltpu.SemaphoreType.DMA,
            pltpu.SemaphoreType.DMA,
        ],
    )
    def sc_gather(m_hbm, a_flat_hbm, idx_hbm, tgt_hbm, out_hbm, part_hbm,
                  idx_v, tgt_v, fidx_v, nll_v, rows_v, acc_v, sem_r, sem_n):
        wid = lax.axis_index("s") * NC + lax.axis_index("c")
        base = wid * b_per_w
        pltpu.sync_copy(idx_hbm.at[pl.ds(base, b_per_w)], idx_v)
        pltpu.sync_copy(tgt_hbm.at[pl.ds(base, b_per_w)], tgt_v)

        def body(c, acc):
            off = c * CH
            rows_dma = pltpu.async_copy(
                m_hbm.at[idx_v.at[pl.ds(off, CH)]], rows_v, sem_r)
            for k in range(CH // L):
                i16 = idx_v[pl.ds(off + k * L, L)]
                t16 = tgt_v[pl.ds(off + k * L, L)]
                fidx_v[pl.ds(k * L, L)] = i16 * V + t16
            nll_dma = pltpu.async_copy(a_flat_hbm.at[fidx_v], nll_v, sem_n)
            nll_dma.wait()
            for k in range(CH // L):
                acc = acc + nll_v[pl.ds(k * L, L)]
            rows_dma.wait()
            pltpu.sync_copy(rows_v, out_hbm.at[pl.ds(base + off, CH)])
            return acc

        acc = lax.fori_loop(0, n_ch, body, jnp.zeros((L,), jnp.float32))
        acc_v[...] = acc
        pltpu.sync_copy(acc_v, part_hbm.at[wid])

    return sc_gather


def kernel(idx, targets, token_table, pos_table, W, b):
    B, T = idx.shape
    n = B * T
    m, a = _tc_tables(token_table, W, b)
    idx_f = idx.reshape(n).astype(jnp.int32)
    tgt_f = targets.reshape(n).astype(jnp.int32)
    logits_flat, parts = _make_sc_gather(n)(
        m, a.reshape(V * V), idx_f, tgt_f)
    loss = jnp.sum(parts) / n
    return (logits_flat, loss)


# speedup vs baseline: 1.4126x; 1.0063x over previous
import functools

import jax
import jax.numpy as jnp
from jax import lax
from jax.experimental import pallas as pl
from jax.experimental.pallas import tpu as pltpu
from jax.experimental.pallas import tpu_sc as plsc

V = 1000
NC = 2
NS = 16
NW = NC * NS
L = 16
CH = 32


def _tc_tables_body(tt_ref, w_ref, b_ref, m_ref, a_ref):
    m = jnp.dot(tt_ref[...], w_ref[...], preferred_element_type=jnp.float32)
    m = m + b_ref[...]
    m_ref[...] = m
    mx = jnp.max(m, axis=1, keepdims=True)
    s = jnp.sum(jnp.exp(m - mx), axis=1, keepdims=True)
    a_ref[...] = (mx + jnp.log(s)) - m


def _tc_tables(token_table, w, b):
    return pl.pallas_call(
        _tc_tables_body,
        out_shape=(
            jax.ShapeDtypeStruct((V, V), jnp.float32),
            jax.ShapeDtypeStruct((V, V), jnp.float32),
        ),
    )(token_table, w, b.reshape(1, V))


@functools.lru_cache(maxsize=None)
def _make_sc_gather(n_rows):
    b_per_w = n_rows // NW
    n_ch = b_per_w // CH
    n_pair = n_ch // 2
    mesh = plsc.VectorSubcoreMesh(core_axis_name="c", subcore_axis_name="s")

    @functools.partial(
        pl.kernel,
        out_type=(
            jax.ShapeDtypeStruct((n_rows, V), jnp.float32),
            jax.ShapeDtypeStruct((NW, L), jnp.float32),
        ),
        mesh=mesh,
        compiler_params=pltpu.CompilerParams(use_tc_tiling_on_sc=False),
        scratch_types=[
            pltpu.VMEM((b_per_w,), jnp.int32),
            pltpu.VMEM((b_per_w,), jnp.int32),
            pltpu.VMEM((CH,), jnp.int32),
            pltpu.VMEM((CH,), jnp.int32),
            pltpu.VMEM((CH,), jnp.float32),
            pltpu.VMEM((CH,), jnp.float32),
            pltpu.VMEM((CH, V), jnp.float32),
            pltpu.VMEM((CH, V), jnp.float32),
            pltpu.VMEM((L,), jnp.float32),
            pltpu.SemaphoreType.DMA,
            pltpu.SemaphoreType.DMA,
            pltpu.SemaphoreType.DMA,
            pltpu.SemaphoreType.DMA,
            pltpu.SemaphoreType.DMA,
            pltpu.SemaphoreType.DMA,
        ],
    )
    def sc_gather(m_hbm, a_flat_hbm, idx_hbm, tgt_hbm, out_hbm, part_hbm,
                  idx_v, tgt_v, fidx0, fidx1, nll0, nll1, rows0, rows1,
                  acc_v, gsem0, gsem1, nsem0, nsem1, osem0, osem1):
        wid = lax.axis_index("s") * NC + lax.axis_index("c")
        base = wid * b_per_w
        pltpu.sync_copy(idx_hbm.at[pl.ds(base, b_per_w)], idx_v)
        pltpu.sync_copy(tgt_hbm.at[pl.ds(base, b_per_w)], tgt_v)

        def fire(off, fidx, nll, rows, gsem, nsem):
            pltpu.async_copy(m_hbm.at[idx_v.at[pl.ds(off, CH)]], rows, gsem)
            for k in range(CH // L):
                i16 = idx_v[pl.ds(off + k * L, L)]
                t16 = tgt_v[pl.ds(off + k * L, L)]
                fidx[pl.ds(k * L, L)] = i16 * V + t16
            pltpu.async_copy(a_flat_hbm.at[fidx], nll, nsem)

        def wait_rows(off, rows, gsem):
            pltpu.make_async_copy(
                m_hbm.at[idx_v.at[pl.ds(off, CH)]], rows, gsem).wait()

        def wait_out(off, rows, osem):
            pltpu.make_async_copy(
                rows, out_hbm.at[pl.ds(base + off, CH)], osem).wait()

        fire(0, fidx0, nll0, rows0, gsem0, nsem0)
        fire(CH, fidx1, nll1, rows1, gsem1, nsem1)

        def body(i, acc):
            a_off = (2 * i) * CH
            b_off = a_off + CH
            last = i >= n_pair - 1

            wait_rows(a_off, rows0, gsem0)
            pltpu.make_async_copy(a_flat_hbm.at[fidx0], nll0, nsem0).wait()
            for k in range(CH // L):
                acc = acc + nll0[pl.ds(k * L, L)]
            pltpu.async_copy(rows0, out_hbm.at[pl.ds(base + a_off, CH)], osem0)

            wait_rows(b_off, rows1, gsem1)
            pltpu.make_async_copy(a_flat_hbm.at[fidx1], nll1, nsem1).wait()
            for k in range(CH // L):
                acc = acc + nll1[pl.ds(k * L, L)]
            pltpu.async_copy(rows1, out_hbm.at[pl.ds(base + b_off, CH)], osem1)

            wait_out(a_off, rows0, osem0)
            @pl.when(jnp.logical_not(last))
            def _():
                fire(a_off + 2 * CH, fidx0, nll0, rows0, gsem0, nsem0)

            wait_out(b_off, rows1, osem1)
            @pl.when(jnp.logical_not(last))
            def _():
                fire(b_off + 2 * CH, fidx1, nll1, rows1, gsem1, nsem1)

            return acc

        acc = lax.fori_loop(0, n_pair, body, jnp.zeros((L,), jnp.float32))
        acc_v[...] = acc
        pltpu.sync_copy(acc_v, part_hbm.at[wid])

    return sc_gather


def kernel(idx, targets, token_table, pos_table, W, b):
    B, T = idx.shape
    n = B * T
    m, a = _tc_tables(token_table, W, b)
    idx_f = idx.reshape(n).astype(jnp.int32)
    tgt_f = targets.reshape(n).astype(jnp.int32)
    logits_flat, parts = _make_sc_gather(n)(
        m, a.reshape(V * V), idx_f, tgt_f)
    loss = jnp.sum(parts) / n
    return (logits_flat, loss)


# speedup vs baseline: 2.0873x; 1.4777x over previous
import functools

import jax
import jax.numpy as jnp
from jax import lax
from jax.experimental import pallas as pl
from jax.experimental.pallas import tpu as pltpu
from jax.experimental.pallas import tpu_sc as plsc

V = 1000
VP = 1024
VA = 896
VT = V - VA
NC = 2
NS = 16
NW = NC * NS
L = 16
CH = 32
RB = 8192


def _tc_tables_body(tt_ref, w_ref, b_ref, ma_ref, mt_ref, a_ref):
    m = jnp.dot(tt_ref[...], w_ref[...], preferred_element_type=jnp.float32)
    m = m + b_ref[...]
    core = m[:, :V]
    ma_ref[...] = m[:, :VA]
    mt_ref[...] = m[:, VA:VA + 128]
    mx = jnp.max(core, axis=1, keepdims=True)
    s = jnp.sum(jnp.exp(core - mx), axis=1, keepdims=True)
    a_ref[...] = (mx + jnp.log(s)) - core


def _tc_tables(token_table, w, b):
    wp = jnp.pad(w, ((0, 0), (0, VP - V)))
    bp = jnp.pad(b, (0, VP - V)).reshape(1, VP)
    return pl.pallas_call(
        _tc_tables_body,
        out_shape=(
            jax.ShapeDtypeStruct((V, VA), jnp.float32),
            jax.ShapeDtypeStruct((V, 128), jnp.float32),
            jax.ShapeDtypeStruct((V, V), jnp.float32),
        ),
    )(token_table, wp, bp)


def _tail_body(o_in_ref, idx_ref, mt_ref, o_ref):
    del o_in_ref
    ids = idx_ref[0, 0, :]
    onehot = (ids[:, None] == lax.broadcasted_iota(jnp.int32, (RB, V), 1))
    sel = onehot.astype(jnp.float32)
    o_ref[...] = jnp.dot(sel, mt_ref[...], preferred_element_type=jnp.float32)


def _tail_fill(out1, idx_f, m_tail, n_rows):
    n_blk = n_rows // RB
    return pl.pallas_call(
        _tail_body,
        grid=(n_blk,),
        in_specs=[
            pl.BlockSpec((RB, 128), lambda g: (g, VA // 128)),
            pl.BlockSpec((1, 1, RB), lambda g: (g, 0, 0)),
            pl.BlockSpec((V, 128), lambda g: (0, 0)),
        ],
        out_specs=pl.BlockSpec((RB, 128), lambda g: (g, VA // 128)),
        out_shape=jax.ShapeDtypeStruct((n_rows, V), jnp.float32),
        input_output_aliases={0: 0},
    )(out1, idx_f.reshape(n_blk, 1, RB), m_tail)


@functools.lru_cache(maxsize=None)
def _make_sc_gather(n_rows):
    b_per_w = n_rows // NW
    n_ch = b_per_w // CH
    n_pair = n_ch // 2
    mesh = plsc.VectorSubcoreMesh(core_axis_name="c", subcore_axis_name="s")

    @functools.partial(
        pl.kernel,
        out_type=(
            jax.ShapeDtypeStruct((n_rows, V), jnp.float32),
            jax.ShapeDtypeStruct((NW * L,), jnp.float32),
        ),
        mesh=mesh,
        compiler_params=pltpu.CompilerParams(use_tc_tiling_on_sc=True),
        scratch_types=[
            pltpu.VMEM((b_per_w,), jnp.int32),
            pltpu.VMEM((b_per_w,), jnp.int32),
            pltpu.VMEM((CH,), jnp.int32),
            pltpu.VMEM((CH,), jnp.int32),
            pltpu.VMEM((CH,), jnp.float32),
            pltpu.VMEM((CH,), jnp.float32),
            pltpu.VMEM((CH, VA), jnp.float32),
            pltpu.VMEM((CH, VA), jnp.float32),
            pltpu.VMEM((L,), jnp.float32),
            pltpu.SemaphoreType.DMA,
            pltpu.SemaphoreType.DMA,
            pltpu.SemaphoreType.DMA,
            pltpu.SemaphoreType.DMA,
            pltpu.SemaphoreType.DMA,
            pltpu.SemaphoreType.DMA,
        ],
    )
    def sc_gather(m_hbm, a_flat_hbm, idx_hbm, tgt_hbm, out_hbm, part_hbm,
                  idx_v, tgt_v, fidx0, fidx1, nll0, nll1, rows0, rows1,
                  acc_v, gsem0, gsem1, nsem0, nsem1, osem0, osem1):
        wid = lax.axis_index("s") * NC + lax.axis_index("c")
        base = wid * b_per_w
        pltpu.sync_copy(idx_hbm.at[pl.ds(base, b_per_w)], idx_v)
        pltpu.sync_copy(tgt_hbm.at[pl.ds(base, b_per_w)], tgt_v)

        def fire(off, fidx, nll, rows, gsem, nsem):
            pltpu.async_copy(m_hbm.at[idx_v.at[pl.ds(off, CH)]], rows, gsem)
            for k in range(CH // L):
                i16 = idx_v[pl.ds(off + k * L, L)]
                t16 = tgt_v[pl.ds(off + k * L, L)]
                fidx[pl.ds(k * L, L)] = i16 * V + t16
            pltpu.async_copy(a_flat_hbm.at[fidx], nll, nsem)

        def wait_rows(off, rows, gsem):
            pltpu.make_async_copy(
                m_hbm.at[idx_v.at[pl.ds(off, CH)]], rows, gsem).wait()

        def out_dst(off):
            return out_hbm.at[pl.ds(base + off, CH), pl.ds(0, VA)]

        fire(0, fidx0, nll0, rows0, gsem0, nsem0)
        fire(CH, fidx1, nll1, rows1, gsem1, nsem1)

        def body(i, acc):
            a_off = (2 * i) * CH
            b_off = a_off + CH
            last = i >= n_pair - 1

            wait_rows(a_off, rows0, gsem0)
            pltpu.make_async_copy(a_flat_hbm.at[fidx0], nll0, nsem0).wait()
            for k in range(CH // L):
                acc = acc + nll0[pl.ds(k * L, L)]
            pltpu.async_copy(rows0, out_dst(a_off), osem0)

            wait_rows(b_off, rows1, gsem1)
            pltpu.make_async_copy(a_flat_hbm.at[fidx1], nll1, nsem1).wait()
            for k in range(CH // L):
                acc = acc + nll1[pl.ds(k * L, L)]
            pltpu.async_copy(rows1, out_dst(b_off), osem1)

            pltpu.make_async_copy(rows0, out_dst(a_off), osem0).wait()
            @pl.when(jnp.logical_not(last))
            def _():
                fire(a_off + 2 * CH, fidx0, nll0, rows0, gsem0, nsem0)

            pltpu.make_async_copy(rows1, out_dst(b_off), osem1).wait()
            @pl.when(jnp.logical_not(last))
            def _():
                fire(b_off + 2 * CH, fidx1, nll1, rows1, gsem1, nsem1)

            return acc

        acc = lax.fori_loop(0, n_pair, body, jnp.zeros((L,), jnp.float32))
        acc_v[...] = acc
        pltpu.sync_copy(acc_v, part_hbm.at[pl.ds(wid * L, L)])

    return sc_gather


def kernel(idx, targets, token_table, pos_table, W, b):
    B, T = idx.shape
    n = B * T
    m_main, m_tail, a = _tc_tables(token_table, W, b)
    idx_f = idx.reshape(n).astype(jnp.int32)
    tgt_f = targets.reshape(n).astype(jnp.int32)
    out1, parts = _make_sc_gather(n)(
        m_main, a.reshape(V * V), idx_f, tgt_f)
    logits_flat = _tail_fill(out1, idx_f, m_tail, n)
    loss = jnp.sum(parts) / n
    return (logits_flat, loss)
